# baseline (speedup 1.0000x reference)
import functools

import jax
import jax.numpy as jnp
from jax import lax
from jax.experimental import pallas as pl
from jax.experimental.pallas import tpu as pltpu
from jax.experimental.pallas import tpu_sc as plsc

BATCH = 128
SEQ = 512
D = 64
NC = 2
NS = 16
NW = NC * NS
ROWS_PER_W = BATCH * SEQ // NW
CHUNK = 128
NCHUNK = ROWS_PER_W // CHUNK


def _sc_gather(idx3, table):
    mesh = plsc.VectorSubcoreMesh(core_axis_name="c", subcore_axis_name="s")

    @functools.partial(
        pl.kernel,
        out_type=jax.ShapeDtypeStruct((BATCH * SEQ, D), jnp.float32),
        mesh=mesh,
        scratch_types=[
            pltpu.VMEM((NCHUNK, CHUNK), jnp.int32),
            pltpu.VMEM((CHUNK, D), jnp.float32),
            pltpu.VMEM((CHUNK, D), jnp.float32),
            pltpu.SemaphoreType.DMA,
            pltpu.SemaphoreType.DMA,
        ],
        compiler_params=pltpu.CompilerParams(use_tc_tiling_on_sc=False),
    )
    def k(idx_hbm, table_hbm, out_hbm, idx_v, rows0, rows1, sem0, sem1):
        wid = lax.axis_index("s") * NC + lax.axis_index("c")
        base = wid * ROWS_PER_W
        pltpu.sync_copy(idx_hbm.at[wid], idx_v)
        bufs = (rows0, rows1)
        sems = (sem0, sem1)
        handles = [None, None]
        handles[0] = pltpu.async_copy(table_hbm.at[idx_v.at[0]], bufs[0], sems[0])
        for j in range(NCHUNK):
            b = j % 2
            nb = (j + 1) % 2
            if j + 1 < NCHUNK:
                handles[nb] = pltpu.async_copy(
                    table_hbm.at[idx_v.at[j + 1]], bufs[nb], sems[nb])
            handles[b].wait()
            pltpu.sync_copy(bufs[b], out_hbm.at[pl.ds(base + j * CHUNK, CHUNK)])

    return k(idx3, table)


def _tc_add_relu(g, w_pos):
    BB = 8

    def body(g_ref, p_ref, o_ref):
        o_ref[...] = jnp.maximum(g_ref[...] + p_ref[...][None], 0.0)

    return pl.pallas_call(
        body,
        grid=(BATCH // BB,),
        in_specs=[
            pl.BlockSpec((BB, SEQ, D), lambda i: (i, 0, 0)),
            pl.BlockSpec((SEQ, D), lambda i: (0, 0)),
        ],
        out_specs=pl.BlockSpec((BB, SEQ, D), lambda i: (i, 0, 0)),
        out_shape=jax.ShapeDtypeStruct((BATCH, SEQ, D), jnp.float32),
    )(g, w_pos)


def kernel(X, W_word, W_pos):
    idx3 = X.astype(jnp.int32).reshape(NW, NCHUNK, CHUNK)
    g = _sc_gather(idx3, W_word).reshape(BATCH, SEQ, D)
    return _tc_add_relu(g, W_pos)
